# Initial kernel scaffold; baseline (speedup 1.0000x reference)
import jax
import jax.numpy as jnp
from jax.experimental import pallas as pl


def kernel(x, edge_index, W, a):
    raise NotImplementedError("write your pallas kernel here")


# speedup vs baseline: 61.2265x; 61.2265x over previous
import functools
import jax
import jax.numpy as jnp
from jax import lax
from jax.experimental import pallas as pl
from jax.experimental.pallas import tpu as pltpu
from jax.experimental.pallas import tpu_sc as plsc

N_NODES = 10000
N_EDGES = 320000
IN_F = 128
OUT_F = 32
HEADS = 4
NEG_SLOPE = 0.2

NC = 2
NS = 16
NW = NC * NS
EPT = N_EDGES // NW
CHUNK = 80
NCHUNK = EPT // CHUNK
ROW = IN_F + 16
RPT = 624

BN = 1000


def _proj_body(x_ref, wc_ref, amat_ref, h_ref, st_ref, mx_ref):
    i = pl.program_id(0)
    xb = x_ref[...]
    hb = jnp.dot(xb, wc_ref[...], preferred_element_type=jnp.float32)
    h_ref[...] = hb
    sb = jnp.dot(hb, amat_ref[...], preferred_element_type=jnp.float32)
    st_ref[...] = sb
    bm = jnp.broadcast_to(jnp.max(sb, axis=0)[:, None], (8, 128))

    @pl.when(i == 0)
    def _():
        mx_ref[...] = bm

    @pl.when(i > 0)
    def _():
        mx_ref[...] = jnp.maximum(mx_ref[...], bm)


def _project(x, wc, amat):
    grid = N_NODES // BN
    return pl.pallas_call(
        _proj_body,
        grid=(grid,),
        in_specs=[
            pl.BlockSpec((BN, IN_F), lambda i: (i, 0)),
            pl.BlockSpec((IN_F, IN_F), lambda i: (0, 0)),
            pl.BlockSpec((IN_F, 2 * HEADS), lambda i: (0, 0)),
        ],
        out_specs=[
            pl.BlockSpec((BN, IN_F), lambda i: (i, 0)),
            pl.BlockSpec((BN, 2 * HEADS), lambda i: (i, 0)),
            pl.BlockSpec((8, 128), lambda i: (0, 0)),
        ],
        out_shape=[
            jax.ShapeDtypeStruct((N_NODES, IN_F), jnp.float32),
            jax.ShapeDtypeStruct((N_NODES, 2 * HEADS), jnp.float32),
            jax.ShapeDtypeStruct((8, 128), jnp.float32),
        ],
    )(x, wc, amat)


def _scalar(vec, i):
    return lax.squeeze(lax.slice(vec, (i,), (i + 1,)), (0,))


def _edge_body(h_hbm, st_hbm, src_hbm, dst_hbm, mx_hbm, out_hbm,
               srcb, dstb, rows, msg, srows, drows, mxv, acc, sem):
    cid = lax.axis_index("c")
    sid = lax.axis_index("s")
    w = cid * NS + sid

    pltpu.sync_copy(mx_hbm, mxv)

    zero16 = jnp.zeros((16,), jnp.float32)

    @pl.loop(0, CHUNK)
    def _(r):
        for k in range(ROW // 16):
            msg[r, pl.ds(k * 16, 16)] = zero16

    base = sid * RPT
    for b in range(7):
        pltpu.sync_copy(msg, acc.at[pl.ds(base + b * CHUNK, CHUNK)])
    pltpu.sync_copy(msg.at[pl.ds(0, 64)], acc.at[pl.ds(base + 560, 64)])

    @pl.when(sid == NS - 1)
    def _():
        pltpu.sync_copy(msg.at[pl.ds(0, 16)],
                        acc.at[pl.ds(NS * RPT, 16)])

    plsc.subcore_barrier()

    mxvec = mxv[...]
    mb = []
    for h in range(HEADS):
        t = _scalar(mxvec, h) + _scalar(mxvec, HEADS + h)
        mb.append(jnp.where(t > 0.0, t, NEG_SLOPE * t))

    @pl.loop(0, NCHUNK)
    def _(c):
        ebase = pl.multiple_of(w * EPT + c * CHUNK, CHUNK)
        pltpu.sync_copy(src_hbm.at[pl.ds(ebase, CHUNK)], srcb)
        pltpu.sync_copy(dst_hbm.at[pl.ds(ebase, CHUNK)], dstb)
        pltpu.async_copy(h_hbm.at[srcb], rows, sem).wait()
        pltpu.sync_copy(st_hbm.at[srcb], srows)
        pltpu.sync_copy(st_hbm.at[dstb], drows)

        for g in range(CHUNK // 16):
            rowid = lax.iota(jnp.int32, 16) + (g * 16)
            exs = []
            for h in range(HEADS):
                a_s = plsc.load_gather(
                    srows, [rowid, jnp.full((16,), h, jnp.int32)])
                a_d = plsc.load_gather(
                    drows, [rowid, jnp.full((16,), HEADS + h, jnp.int32)])
                e = a_s + a_d
                e = jnp.where(e > 0.0, e, NEG_SLOPE * e)
                ex = jnp.exp(e - mb[h])
                exs.append(ex)
                plsc.store_scatter(
                    msg, [rowid, jnp.full((16,), IN_F + h, jnp.int32)], ex)
            for jj in range(16):
                j = g * 16 + jj
                for h in range(HEADS):
                    exj = _scalar(exs[h], jj)
                    msg[j, pl.ds(h * 32, 16)] = rows[j, pl.ds(h * 32, 16)] * exj
                    msg[j, pl.ds(h * 32 + 16, 16)] = (
                        rows[j, pl.ds(h * 32 + 16, 16)] * exj)

        pltpu.sync_copy(msg, acc.at[dstb], add=True)

    plsc.subcore_barrier()
    pltpu.sync_copy(acc.at[pl.ds(sid * RPT, RPT)],
                    out_hbm.at[cid, pl.ds(sid * RPT, RPT)])

    @pl.when(sid == NS - 1)
    def _():
        pltpu.sync_copy(acc.at[pl.ds(NS * RPT, N_NODES - NS * RPT)],
                        out_hbm.at[cid, pl.ds(NS * RPT, N_NODES - NS * RPT)])


def _edge_aggregate(h_all, st16, src, dst, mx16):
    mesh = plsc.VectorSubcoreMesh(core_axis_name="c", subcore_axis_name="s")
    import dataclasses
    cp = pltpu.CompilerParams()
    fields = pltpu.CompilerParams.__dataclass_fields__
    if "needs_layout_passes" in fields:
        cp = dataclasses.replace(cp, needs_layout_passes=False)
    if "use_tc_tiling_on_sc" in fields:
        cp = dataclasses.replace(cp, use_tc_tiling_on_sc=False)
    run = pl.kernel(
        _edge_body,
        out_type=jax.ShapeDtypeStruct((NC, N_NODES, ROW), jnp.float32),
        mesh=mesh,
        compiler_params=cp,
        scratch_types=[
            pltpu.VMEM((CHUNK,), jnp.int32),
            pltpu.VMEM((CHUNK,), jnp.int32),
            pltpu.VMEM((CHUNK, IN_F), jnp.float32),
            pltpu.VMEM((CHUNK, ROW), jnp.float32),
            pltpu.VMEM((CHUNK, 16), jnp.float32),
            pltpu.VMEM((CHUNK, 16), jnp.float32),
            pltpu.VMEM((16,), jnp.float32),
            pltpu.VMEM_SHARED((N_NODES, ROW), jnp.float32),
            pltpu.SemaphoreType.DMA,
        ],
    )
    return run(h_all, st16, src, dst, mx16)


def _fin_body(pa_ref, pb_ref, o_ref):
    y = pa_ref[...] + pb_ref[...]
    cols = []
    for h in range(HEADS):
        d = y[:, IN_F + h:IN_F + h + 1] + 1e-16
        cols.append(y[:, h * 32:(h + 1) * 32] / d)
    r = jnp.concatenate(cols, axis=1)
    o_ref[...] = jnp.where(r > 0.0, r, jnp.exp(r) - 1.0)


def _finalize(pa, pb):
    grid = N_NODES // BN
    return pl.pallas_call(
        _fin_body,
        grid=(grid,),
        in_specs=[
            pl.BlockSpec((BN, ROW), lambda i: (i, 0)),
            pl.BlockSpec((BN, ROW), lambda i: (i, 0)),
        ],
        out_specs=pl.BlockSpec((BN, IN_F), lambda i: (i, 0)),
        out_shape=jax.ShapeDtypeStruct((N_NODES, IN_F), jnp.float32),
    )(pa, pb)


def kernel(x, edge_index, W, a):
    wc = W.transpose(1, 0, 2).reshape(IN_F, HEADS * OUT_F)
    amat = jnp.zeros((IN_F, 2 * HEADS), jnp.float32)
    for h in range(HEADS):
        amat = amat.at[h * OUT_F:(h + 1) * OUT_F, h].set(a[h, :OUT_F])
        amat = amat.at[h * OUT_F:(h + 1) * OUT_F, HEADS + h].set(a[h, OUT_F:])

    h_all, st, mx = _project(x, wc, amat)

    st16 = jnp.concatenate(
        [st, jnp.zeros((N_NODES, 8), jnp.float32)], axis=1)
    mx16 = jnp.concatenate([mx[:, 0], jnp.zeros((8,), jnp.float32)])

    partials = _edge_aggregate(h_all, st16, edge_index[0], edge_index[1], mx16)
    return _finalize(partials[0], partials[1])


# speedup vs baseline: 95.8480x; 1.5655x over previous
import dataclasses
import functools
import jax
import jax.numpy as jnp
from jax import lax
from jax.experimental import pallas as pl
from jax.experimental.pallas import tpu as pltpu
from jax.experimental.pallas import tpu_sc as plsc

N_NODES = 10000
N_EDGES = 320000
IN_F = 128
OUT_F = 32
HEADS = 4
NEG_SLOPE = 0.2

NC = 2
NS = 16
NW = NC * NS
EPT = N_EDGES // NW
CHUNK = 80
NCHUNK = EPT // CHUNK
ROW = IN_F + 16
RPT = 624

BN = 1000


def _proj_body(x_ref, wc_ref, amat_ref, h_ref, st_ref, mx_ref):
    i = pl.program_id(0)
    xb = x_ref[...]
    hb = jnp.dot(xb, wc_ref[...], preferred_element_type=jnp.float32)
    h_ref[...] = hb
    sb = jnp.dot(hb, amat_ref[...], preferred_element_type=jnp.float32)
    st_ref[...] = sb
    bm = jnp.broadcast_to(jnp.max(sb, axis=0)[:, None], (8, 128))

    @pl.when(i == 0)
    def _():
        mx_ref[...] = bm

    @pl.when(i > 0)
    def _():
        mx_ref[...] = jnp.maximum(mx_ref[...], bm)


def _project(x, wc, amat):
    grid = N_NODES // BN
    return pl.pallas_call(
        _proj_body,
        grid=(grid,),
        in_specs=[
            pl.BlockSpec((BN, IN_F), lambda i: (i, 0)),
            pl.BlockSpec((IN_F, IN_F), lambda i: (0, 0)),
            pl.BlockSpec((IN_F, 2 * HEADS), lambda i: (0, 0)),
        ],
        out_specs=[
            pl.BlockSpec((BN, IN_F), lambda i: (i, 0)),
            pl.BlockSpec((BN, 2 * HEADS), lambda i: (i, 0)),
            pl.BlockSpec((8, 128), lambda i: (0, 0)),
        ],
        out_shape=[
            jax.ShapeDtypeStruct((N_NODES, IN_F), jnp.float32),
            jax.ShapeDtypeStruct((N_NODES, 2 * HEADS), jnp.float32),
            jax.ShapeDtypeStruct((8, 128), jnp.float32),
        ],
    )(x, wc, amat)


def _scalar(vec, i):
    return lax.squeeze(lax.slice(vec, (i,), (i + 1,)), (0,))


def _edge_body(h_hbm, st_hbm, ei_hbm, mx_hbm, out_hbm,
               idxa, idxb, rowsa, rowsb, srowsa, srowsb, drowsa, drowsb,
               msg, mxv, acc, isema, isemb, gsema, gsemb):
    cid = lax.axis_index("c")
    sid = lax.axis_index("s")
    w = cid * NS + sid

    pltpu.sync_copy(mx_hbm, mxv)

    zero16 = jnp.zeros((16,), jnp.float32)

    @pl.loop(0, CHUNK)
    def _(r):
        for k in range(ROW // 16):
            msg[r, pl.ds(k * 16, 16)] = zero16

    if True:
        base = sid * RPT
        for b in range(7):
            pltpu.sync_copy(msg, acc.at[pl.ds(base + b * CHUNK, CHUNK)])
        pltpu.sync_copy(msg.at[pl.ds(0, 64)], acc.at[pl.ds(base + 560, 64)])

        @pl.when(sid == NS - 1)
        def _():
            pltpu.sync_copy(msg.at[pl.ds(0, 16)],
                            acc.at[pl.ds(NS * RPT, 16)])

        plsc.subcore_barrier()

        mxvec = mxv[...]
        mb = []
        for h in range(HEADS):
            t = _scalar(mxvec, h) + _scalar(mxvec, HEADS + h)
            mb.append(jnp.where(t > 0.0, t, NEG_SLOPE * t))

        def issue_idx(c, idx, isem):
            pltpu.async_copy(ei_hbm.at[w, c], idx, isem)

        def wait_idx(c, idx, isem):
            pltpu.make_async_copy(ei_hbm.at[w, c], idx, isem).wait()

        def issue_gather(c, idx, rows, srows, drows, gsem):
            pltpu.async_copy(h_hbm.at[idx.at[0]], rows, gsem)
            pltpu.async_copy(st_hbm.at[idx.at[0]], srows, gsem)
            pltpu.async_copy(st_hbm.at[idx.at[1]], drows, gsem)

        def wait_gather(c, idx, rows, srows, drows, gsem):
            pltpu.make_async_copy(h_hbm.at[idx.at[0]], rows, gsem).wait()
            pltpu.make_async_copy(st_hbm.at[idx.at[0]], srows, gsem).wait()
            pltpu.make_async_copy(st_hbm.at[idx.at[1]], drows, gsem).wait()

        def compute_scatter(idx, rows, srows, drows):
            for g in range(CHUNK // 16):
                rowid = lax.iota(jnp.int32, 16) + (g * 16)
                exs = []
                for h in range(HEADS):
                    a_s = plsc.load_gather(
                        srows, [rowid, jnp.full((16,), h, jnp.int32)])
                    a_d = plsc.load_gather(
                        drows, [rowid, jnp.full((16,), HEADS + h, jnp.int32)])
                    e = a_s + a_d
                    e = jnp.where(e > 0.0, e, NEG_SLOPE * e)
                    ex = jnp.exp(e - mb[h])
                    exs.append(ex)
                    plsc.store_scatter(
                        msg, [rowid, jnp.full((16,), IN_F + h, jnp.int32)], ex)
                for jj in range(16):
                    j = g * 16 + jj
                    for h in range(HEADS):
                        exj = _scalar(exs[h], jj)
                        msg[j, pl.ds(h * 32, 16)] = (
                            rows[j, pl.ds(h * 32, 16)] * exj)
                        msg[j, pl.ds(h * 32 + 16, 16)] = (
                            rows[j, pl.ds(h * 32 + 16, 16)] * exj)
            pltpu.sync_copy(msg, acc.at[idx.at[1]], add=True)

        seta = (idxa, rowsa, srowsa, drowsa, gsema)
        setb = (idxb, rowsb, srowsb, drowsb, gsemb)

        issue_idx(0, idxa, isema)
        wait_idx(0, idxa, isema)
        issue_gather(0, *seta)
        issue_idx(1, idxb, isemb)

        @pl.loop(0, NCHUNK - 1, step=2)
        def _(c):
            wait_gather(c, *seta)
            wait_idx(c + 1, idxb, isemb)
            issue_gather(c + 1, *setb)
            compute_scatter(idxa, rowsa, srowsa, drowsa)
            issue_idx(c + 2, idxa, isema)
            wait_gather(c + 1, *setb)
            wait_idx(c + 2, idxa, isema)
            issue_gather(c + 2, *seta)
            compute_scatter(idxb, rowsb, srowsb, drowsb)
            issue_idx(c + 3, idxb, isemb)

        wait_gather(NCHUNK - 1, *seta)
        compute_scatter(idxa, rowsa, srowsa, drowsa)
        wait_idx(NCHUNK, idxb, isemb)

        plsc.subcore_barrier()
        pltpu.sync_copy(acc.at[pl.ds(sid * RPT, RPT)],
                        out_hbm.at[cid, pl.ds(sid * RPT, RPT)])

        @pl.when(sid == NS - 1)
        def _():
            pltpu.sync_copy(acc.at[pl.ds(NS * RPT, N_NODES - NS * RPT)],
                            out_hbm.at[cid, pl.ds(NS * RPT, N_NODES - NS * RPT)])


def _edge_aggregate(h_all, st16, ei_pad, mx16):
    mesh = plsc.VectorSubcoreMesh(core_axis_name="c", subcore_axis_name="s")
    cp = pltpu.CompilerParams()
    fields = pltpu.CompilerParams.__dataclass_fields__
    if "needs_layout_passes" in fields:
        cp = dataclasses.replace(cp, needs_layout_passes=False)
    if "use_tc_tiling_on_sc" in fields:
        cp = dataclasses.replace(cp, use_tc_tiling_on_sc=False)
    run = pl.kernel(
        _edge_body,
        out_type=jax.ShapeDtypeStruct((NC, N_NODES, ROW), jnp.float32),
        mesh=mesh,
        compiler_params=cp,
        scratch_types=[
            pltpu.VMEM((2, CHUNK), jnp.int32),
            pltpu.VMEM((2, CHUNK), jnp.int32),
            pltpu.VMEM((CHUNK, IN_F), jnp.float32),
            pltpu.VMEM((CHUNK, IN_F), jnp.float32),
            pltpu.VMEM((CHUNK, 16), jnp.float32),
            pltpu.VMEM((CHUNK, 16), jnp.float32),
            pltpu.VMEM((CHUNK, 16), jnp.float32),
            pltpu.VMEM((CHUNK, 16), jnp.float32),
            pltpu.VMEM((CHUNK, ROW), jnp.float32),
            pltpu.VMEM((16,), jnp.float32),
            pltpu.VMEM_SHARED((N_NODES, ROW), jnp.float32),
            pltpu.SemaphoreType.DMA,
            pltpu.SemaphoreType.DMA,
            pltpu.SemaphoreType.DMA,
            pltpu.SemaphoreType.DMA,
        ],
    )
    return run(h_all, st16, ei_pad, mx16)


def _fin_body(pa_ref, pb_ref, o_ref):
    y = pa_ref[...] + pb_ref[...]
    cols = []
    for h in range(HEADS):
        d = y[:, IN_F + h:IN_F + h + 1] + 1e-16
        cols.append(y[:, h * 32:(h + 1) * 32] / d)
    r = jnp.concatenate(cols, axis=1)
    o_ref[...] = jnp.where(r > 0.0, r, jnp.exp(r) - 1.0)


def _finalize(pa, pb):
    grid = N_NODES // BN
    return pl.pallas_call(
        _fin_body,
        grid=(grid,),
        in_specs=[
            pl.BlockSpec((BN, ROW), lambda i: (i, 0)),
            pl.BlockSpec((BN, ROW), lambda i: (i, 0)),
        ],
        out_specs=pl.BlockSpec((BN, IN_F), lambda i: (i, 0)),
        out_shape=jax.ShapeDtypeStruct((N_NODES, IN_F), jnp.float32),
    )(pa, pb)


def kernel(x, edge_index, W, a):
    wc = W.transpose(1, 0, 2).reshape(IN_F, HEADS * OUT_F)
    amat = jnp.zeros((IN_F, 2 * HEADS), jnp.float32)
    for h in range(HEADS):
        amat = amat.at[h * OUT_F:(h + 1) * OUT_F, h].set(a[h, :OUT_F])
        amat = amat.at[h * OUT_F:(h + 1) * OUT_F, HEADS + h].set(a[h, OUT_F:])

    h_all, st, mx = _project(x, wc, amat)

    st16 = jnp.concatenate(
        [st, jnp.zeros((N_NODES, 8), jnp.float32)], axis=1)
    mx16 = jnp.concatenate([mx[:, 0], jnp.zeros((8,), jnp.float32)])

    ei = jnp.stack([edge_index[0].reshape(NW, NCHUNK, CHUNK),
                    edge_index[1].reshape(NW, NCHUNK, CHUNK)], axis=2)
    ei_pad = jnp.pad(ei, ((0, 0), (0, 2), (0, 0), (0, 0)))

    partials = _edge_aggregate(h_all, st16, ei_pad, mx16)
    return _finalize(partials[0], partials[1])


# speedup vs baseline: 106.6991x; 1.1132x over previous
import dataclasses
import functools
import jax
import jax.numpy as jnp
from jax import lax
from jax.experimental import pallas as pl
from jax.experimental.pallas import tpu as pltpu
from jax.experimental.pallas import tpu_sc as plsc

N_NODES = 10000
N_EDGES = 320000
IN_F = 128
OUT_F = 32
HEADS = 4
NEG_SLOPE = 0.2

NC = 2
NS = 16
NW = NC * NS
EPT = N_EDGES // NW
CHUNK = 80
NCHUNK = EPT // CHUNK
ROW = IN_F + 16
RPT = 624

BN = 1000


def _proj_body(x_ref, wc_ref, amat_ref, h_ref, st_ref, mx_ref):
    i = pl.program_id(0)
    xb = x_ref[...]
    hb = jnp.dot(xb, wc_ref[...], preferred_element_type=jnp.float32)
    h_ref[...] = hb
    sb = jnp.dot(hb, amat_ref[...], preferred_element_type=jnp.float32)
    st_ref[...] = jnp.concatenate(
        [sb, jnp.zeros((BN, 8), jnp.float32)], axis=1)
    bm = jnp.broadcast_to(jnp.max(sb, axis=0)[:, None], (8, 128))

    @pl.when(i == 0)
    def _():
        mx_ref[...] = bm

    @pl.when(i > 0)
    def _():
        mx_ref[...] = jnp.maximum(mx_ref[...], bm)


def _project(x, wc, amat):
    grid = N_NODES // BN
    return pl.pallas_call(
        _proj_body,
        grid=(grid,),
        in_specs=[
            pl.BlockSpec((BN, IN_F), lambda i: (i, 0)),
            pl.BlockSpec((IN_F, IN_F), lambda i: (0, 0)),
            pl.BlockSpec((IN_F, 2 * HEADS), lambda i: (0, 0)),
        ],
        out_specs=[
            pl.BlockSpec((BN, IN_F), lambda i: (i, 0)),
            pl.BlockSpec((BN, 16), lambda i: (i, 0)),
            pl.BlockSpec((8, 128), lambda i: (0, 0)),
        ],
        out_shape=[
            jax.ShapeDtypeStruct((N_NODES, IN_F), jnp.float32),
            jax.ShapeDtypeStruct((N_NODES, 16), jnp.float32),
            jax.ShapeDtypeStruct((8, 128), jnp.float32),
        ],
    )(x, wc, amat)


def _scalar(vec, i):
    return lax.squeeze(lax.slice(vec, (i,), (i + 1,)), (0,))


def _edge_body(h_hbm, st_hbm, ei_hbm, mx_hbm, out_hbm,
               idxa, idxb, rowsa, rowsb, srowsa, srowsb, drowsa, drowsb,
               msg, mxv, acc, isema, isemb, gsema, gsemb):
    cid = lax.axis_index("c")
    sid = lax.axis_index("s")
    w = cid * NS + sid

    pltpu.sync_copy(mx_hbm, mxv)

    zero16 = jnp.zeros((16,), jnp.float32)

    @pl.loop(0, CHUNK)
    def _(r):
        for k in range(ROW // 16):
            msg[r, pl.ds(k * 16, 16)] = zero16

    if True:
        base = sid * RPT
        for b in range(7):
            pltpu.sync_copy(msg, acc.at[pl.ds(base + b * CHUNK, CHUNK)])
        pltpu.sync_copy(msg.at[pl.ds(0, 64)], acc.at[pl.ds(base + 560, 64)])

        @pl.when(sid == NS - 1)
        def _():
            pltpu.sync_copy(msg.at[pl.ds(0, 16)],
                            acc.at[pl.ds(NS * RPT, 16)])

        plsc.subcore_barrier()

        mxvec = mxv[...]
        mb = []
        for h in range(HEADS):
            t = _scalar(mxvec, h) + _scalar(mxvec, HEADS + h)
            mb.append(jnp.where(t > 0.0, t, NEG_SLOPE * t))

        def _ebase(c):
            cc = jnp.minimum(c, NCHUNK - 1)
            return pl.multiple_of(w * EPT + cc * CHUNK, CHUNK)

        def issue_idx(c, idx, isem):
            pltpu.async_copy(ei_hbm.at[:, pl.ds(_ebase(c), CHUNK)], idx, isem)

        def wait_idx(c, idx, isem):
            pltpu.make_async_copy(
                ei_hbm.at[:, pl.ds(_ebase(c), CHUNK)], idx, isem).wait()

        def issue_gather(c, idx, rows, srows, drows, gsem):
            pltpu.async_copy(h_hbm.at[idx.at[0]], rows, gsem)
            pltpu.async_copy(st_hbm.at[idx.at[0]], srows, gsem)
            pltpu.async_copy(st_hbm.at[idx.at[1]], drows, gsem)

        def wait_gather(c, idx, rows, srows, drows, gsem):
            pltpu.make_async_copy(h_hbm.at[idx.at[0]], rows, gsem).wait()
            pltpu.make_async_copy(st_hbm.at[idx.at[0]], srows, gsem).wait()
            pltpu.make_async_copy(st_hbm.at[idx.at[1]], drows, gsem).wait()

        def compute_scatter(idx, rows, srows, drows):
            for g in range(CHUNK // 16):
                rowid = lax.iota(jnp.int32, 16) + (g * 16)
                exs = []
                for h in range(HEADS):
                    a_s = plsc.load_gather(
                        srows, [rowid, jnp.full((16,), h, jnp.int32)])
                    a_d = plsc.load_gather(
                        drows, [rowid, jnp.full((16,), HEADS + h, jnp.int32)])
                    e = a_s + a_d
                    e = jnp.where(e > 0.0, e, NEG_SLOPE * e)
                    ex = jnp.exp(e - mb[h])
                    exs.append(ex)
                    plsc.store_scatter(
                        msg, [rowid, jnp.full((16,), IN_F + h, jnp.int32)], ex)
                for jj in range(16):
                    j = g * 16 + jj
                    for h in range(HEADS):
                        exj = _scalar(exs[h], jj)
                        msg[j, pl.ds(h * 32, 16)] = (
                            rows[j, pl.ds(h * 32, 16)] * exj)
                        msg[j, pl.ds(h * 32 + 16, 16)] = (
                            rows[j, pl.ds(h * 32 + 16, 16)] * exj)
            pltpu.sync_copy(msg, acc.at[idx.at[1]], add=True)

        seta = (idxa, rowsa, srowsa, drowsa, gsema)
        setb = (idxb, rowsb, srowsb, drowsb, gsemb)

        issue_idx(0, idxa, isema)
        wait_idx(0, idxa, isema)
        issue_gather(0, *seta)
        issue_idx(1, idxb, isemb)

        @pl.loop(0, NCHUNK - 1, step=2)
        def _(c):
            wait_gather(c, *seta)
            wait_idx(c + 1, idxb, isemb)
            issue_gather(c + 1, *setb)
            compute_scatter(idxa, rowsa, srowsa, drowsa)
            issue_idx(c + 2, idxa, isema)
            wait_gather(c + 1, *setb)
            wait_idx(c + 2, idxa, isema)
            issue_gather(c + 2, *seta)
            compute_scatter(idxb, rowsb, srowsb, drowsb)
            issue_idx(c + 3, idxb, isemb)

        wait_gather(NCHUNK - 1, *seta)
        compute_scatter(idxa, rowsa, srowsa, drowsa)
        wait_idx(NCHUNK, idxb, isemb)

        plsc.subcore_barrier()
        pltpu.sync_copy(acc.at[pl.ds(sid * RPT, RPT)],
                        out_hbm.at[cid, pl.ds(sid * RPT, RPT)])

        @pl.when(sid == NS - 1)
        def _():
            pltpu.sync_copy(acc.at[pl.ds(NS * RPT, N_NODES - NS * RPT)],
                            out_hbm.at[cid, pl.ds(NS * RPT, N_NODES - NS * RPT)])


def _edge_aggregate(h_all, st16, edge_index, mx16):
    mesh = plsc.VectorSubcoreMesh(core_axis_name="c", subcore_axis_name="s")
    cp = pltpu.CompilerParams()
    fields = pltpu.CompilerParams.__dataclass_fields__
    if "needs_layout_passes" in fields:
        cp = dataclasses.replace(cp, needs_layout_passes=False)
    if "use_tc_tiling_on_sc" in fields:
        cp = dataclasses.replace(cp, use_tc_tiling_on_sc=False)
    run = pl.kernel(
        _edge_body,
        out_type=jax.ShapeDtypeStruct((NC, N_NODES, ROW), jnp.float32),
        mesh=mesh,
        compiler_params=cp,
        scratch_types=[
            pltpu.VMEM((2, CHUNK), jnp.int32),
            pltpu.VMEM((2, CHUNK), jnp.int32),
            pltpu.VMEM((CHUNK, IN_F), jnp.float32),
            pltpu.VMEM((CHUNK, IN_F), jnp.float32),
            pltpu.VMEM((CHUNK, 16), jnp.float32),
            pltpu.VMEM((CHUNK, 16), jnp.float32),
            pltpu.VMEM((CHUNK, 16), jnp.float32),
            pltpu.VMEM((CHUNK, 16), jnp.float32),
            pltpu.VMEM((CHUNK, ROW), jnp.float32),
            pltpu.VMEM((16,), jnp.float32),
            pltpu.VMEM_SHARED((N_NODES, ROW), jnp.float32),
            pltpu.SemaphoreType.DMA,
            pltpu.SemaphoreType.DMA,
            pltpu.SemaphoreType.DMA,
            pltpu.SemaphoreType.DMA,
        ],
    )
    return run(h_all, st16, edge_index, mx16)


def _fin_body(p_ref, o_ref):
    y = p_ref[0] + p_ref[1]
    cols = []
    for h in range(HEADS):
        d = y[:, IN_F + h:IN_F + h + 1] + 1e-16
        cols.append(y[:, h * 32:(h + 1) * 32] / d)
    r = jnp.concatenate(cols, axis=1)
    o_ref[...] = jnp.where(r > 0.0, r, jnp.exp(r) - 1.0)


def _finalize(partials):
    grid = N_NODES // BN
    return pl.pallas_call(
        _fin_body,
        grid=(grid,),
        in_specs=[
            pl.BlockSpec((2, BN, ROW), lambda i: (0, i, 0)),
        ],
        out_specs=pl.BlockSpec((BN, IN_F), lambda i: (i, 0)),
        out_shape=jax.ShapeDtypeStruct((N_NODES, IN_F), jnp.float32),
    )(partials)


def kernel(x, edge_index, W, a):
    wc = W.transpose(1, 0, 2).reshape(IN_F, HEADS * OUT_F)
    amat = jnp.zeros((IN_F, 2 * HEADS), jnp.float32)
    for h in range(HEADS):
        amat = amat.at[h * OUT_F:(h + 1) * OUT_F, h].set(a[h, :OUT_F])
        amat = amat.at[h * OUT_F:(h + 1) * OUT_F, HEADS + h].set(a[h, OUT_F:])

    h_all, st16, mx = _project(x, wc, amat)
    mx16 = jnp.concatenate([mx[:, 0], jnp.zeros((8,), jnp.float32)])

    partials = _edge_aggregate(h_all, st16, edge_index, mx16)
    return _finalize(partials)


# speedup vs baseline: 122.8941x; 1.1518x over previous
import dataclasses
import functools
import jax
import jax.numpy as jnp
from jax import lax
from jax.experimental import pallas as pl
from jax.experimental.pallas import tpu as pltpu
from jax.experimental.pallas import tpu_sc as plsc

N_NODES = 10000
N_EDGES = 320000
IN_F = 128
OUT_F = 32
HEADS = 4
NEG_SLOPE = 0.2

NC = 2
NS = 16
NW = NC * NS
EPT = N_EDGES // NW
CHUNK = 80
NCHUNK = EPT // CHUNK
ROW = IN_F + 16
RPT = 624

BN = 1000


def _proj_body(x_ref, wc_ref, amat_ref, h_ref, st_ref, mx_ref):
    i = pl.program_id(0)
    xb = x_ref[...]
    hb = jnp.dot(xb, wc_ref[...], preferred_element_type=jnp.float32)
    h_ref[...] = hb
    sb = jnp.dot(hb, amat_ref[...], preferred_element_type=jnp.float32)
    st_ref[...] = jnp.concatenate(
        [sb, jnp.zeros((BN, 8), jnp.float32)], axis=1)
    bm = jnp.broadcast_to(jnp.max(sb, axis=0)[:, None], (8, 128))

    @pl.when(i == 0)
    def _():
        mx_ref[...] = bm

    @pl.when(i > 0)
    def _():
        mx_ref[...] = jnp.maximum(mx_ref[...], bm)


def _project(x, wc, amat):
    grid = N_NODES // BN
    return pl.pallas_call(
        _proj_body,
        grid=(grid,),
        in_specs=[
            pl.BlockSpec((BN, IN_F), lambda i: (i, 0)),
            pl.BlockSpec((IN_F, IN_F), lambda i: (0, 0)),
            pl.BlockSpec((IN_F, 2 * HEADS), lambda i: (0, 0)),
        ],
        out_specs=[
            pl.BlockSpec((BN, IN_F), lambda i: (i, 0)),
            pl.BlockSpec((BN, 16), lambda i: (i, 0)),
            pl.BlockSpec((8, 128), lambda i: (0, 0)),
        ],
        out_shape=[
            jax.ShapeDtypeStruct((N_NODES, IN_F), jnp.float32),
            jax.ShapeDtypeStruct((N_NODES, 16), jnp.float32),
            jax.ShapeDtypeStruct((8, 128), jnp.float32),
        ],
    )(x, wc, amat)


def _scalar(vec, i):
    return lax.squeeze(lax.slice(vec, (i,), (i + 1,)), (0,))


MSGA = 48
MSGB = CHUNK - MSGA


def _edge_body(h_hbm, st_hbm, ei_hbm, mx_hbm, out_hbm,
               idxa, idxb, rowsa, rowsb, srowsa, srowsb, drowsa, drowsb,
               msga, msgb, sidxa, sidxb, mxv, acc,
               isema, isemb, gsema, gsemb, ssema, ssemb):
    cid = lax.axis_index("c")
    sid = lax.axis_index("s")
    w = cid * NS + sid

    pltpu.sync_copy(mx_hbm, mxv)

    zero16 = jnp.zeros((16,), jnp.float32)

    @pl.loop(0, MSGA)
    def _(r):
        for k in range(ROW // 16):
            msga[r, pl.ds(k * 16, 16)] = zero16

    @pl.loop(0, MSGB)
    def _(r):
        for k in range(ROW // 16):
            msgb[r, pl.ds(k * 16, 16)] = zero16

    if True:
        base = sid * RPT
        for b in range(13):
            pltpu.sync_copy(msga, acc.at[pl.ds(base + b * MSGA, MSGA)])

        @pl.when(sid == NS - 1)
        def _():
            pltpu.sync_copy(msga.at[pl.ds(0, 16)],
                            acc.at[pl.ds(NS * RPT, 16)])

        plsc.subcore_barrier()

        mxvec = mxv[...]
        mb = []
        for h in range(HEADS):
            t = _scalar(mxvec, h) + _scalar(mxvec, HEADS + h)
            mb.append(jnp.where(t > 0.0, t, NEG_SLOPE * t))

        def _ebase(c):
            cc = jnp.minimum(c, NCHUNK - 1)
            return pl.multiple_of(w * EPT + cc * CHUNK, CHUNK)

        def issue_idx(c, idx, isem):
            pltpu.async_copy(ei_hbm.at[:, pl.ds(_ebase(c), CHUNK)], idx, isem)

        def wait_idx(c, idx, isem):
            pltpu.make_async_copy(
                ei_hbm.at[:, pl.ds(_ebase(c), CHUNK)], idx, isem).wait()

        def issue_gather(c, idx, rows, srows, drows, gsem):
            pltpu.async_copy(h_hbm.at[idx.at[0]], rows, gsem)
            pltpu.async_copy(st_hbm.at[idx.at[0]], srows, gsem)
            pltpu.async_copy(st_hbm.at[idx.at[1]], drows, gsem)

        def wait_gather(c, idx, rows, srows, drows, gsem):
            pltpu.make_async_copy(h_hbm.at[idx.at[0]], rows, gsem).wait()
            pltpu.make_async_copy(st_hbm.at[idx.at[0]], srows, gsem).wait()
            pltpu.make_async_copy(st_hbm.at[idx.at[1]], drows, gsem).wait()

        def wait_scatter_a():
            pltpu.make_async_copy(msga, acc.at[sidxa], ssema).wait()

        def wait_scatter_b():
            pltpu.make_async_copy(msgb, acc.at[sidxb], ssemb).wait()

        def compute_half(idx, rows, srows, drows, msg, sidx, ssem,
                         g0, g1, roff, first):
            if not first:
                pltpu.make_async_copy(msg, acc.at[sidx], ssem).wait()
            for g in range(g0, g1):
                rowid = lax.iota(jnp.int32, 16) + (g * 16 - roff)
                exs = []
                for h in range(HEADS):
                    a_s = plsc.load_gather(
                        srows,
                        [rowid + roff, jnp.full((16,), h, jnp.int32)])
                    a_d = plsc.load_gather(
                        drows,
                        [rowid + roff, jnp.full((16,), HEADS + h, jnp.int32)])
                    e = a_s + a_d
                    e = jnp.where(e > 0.0, e, NEG_SLOPE * e)
                    ex = jnp.exp(e - mb[h])
                    exs.append(ex)
                    plsc.store_scatter(
                        msg, [rowid, jnp.full((16,), IN_F + h, jnp.int32)], ex)
                for jj in range(16):
                    j = g * 16 + jj
                    for h in range(HEADS):
                        exj = _scalar(exs[h], jj)
                        msg[j - roff, pl.ds(h * 32, 16)] = (
                            rows[j, pl.ds(h * 32, 16)] * exj)
                        msg[j - roff, pl.ds(h * 32 + 16, 16)] = (
                            rows[j, pl.ds(h * 32 + 16, 16)] * exj)
            nrow = (g1 - g0) * 16
            for o in range(0, nrow, 16):
                sidx[pl.ds(o, 16)] = idx[1, pl.ds(roff + o, 16)]
            pltpu.async_copy(msg, acc.at[sidx], ssem, add=True)

        def process(c, ths, oth, first=False):
            idx, rows, srows, drows, gsem = ths
            isem_t = isema if ths is seta else isemb
            isem_o = isemb if ths is seta else isema
            wait_gather(c, *ths)
            wait_idx(c + 1, oth[0], isem_o)
            issue_gather(c + 1, *oth)
            compute_half(idx, rows, srows, drows, msga, sidxa, ssema,
                         0, 3, 0, first)
            compute_half(idx, rows, srows, drows, msgb, sidxb, ssemb,
                         3, 5, MSGA, first)
            issue_idx(c + 2, idx, isem_t)

        seta = (idxa, rowsa, srowsa, drowsa, gsema)
        setb = (idxb, rowsb, srowsb, drowsb, gsemb)

        issue_idx(0, idxa, isema)
        wait_idx(0, idxa, isema)
        issue_gather(0, *seta)
        issue_idx(1, idxb, isemb)

        process(0, seta, setb, first=True)

        @pl.loop(1, NCHUNK - 1, step=2)
        def _(c):
            process(c, setb, seta)
            process(c + 1, seta, setb)

        wait_gather(NCHUNK, *setb)
        wait_idx(NCHUNK + 1, idxa, isema)
        wait_scatter_a()
        wait_scatter_b()

        plsc.subcore_barrier()
        pltpu.sync_copy(acc.at[pl.ds(sid * RPT, RPT)],
                        out_hbm.at[cid, pl.ds(sid * RPT, RPT)])

        @pl.when(sid == NS - 1)
        def _():
            pltpu.sync_copy(acc.at[pl.ds(NS * RPT, N_NODES - NS * RPT)],
                            out_hbm.at[cid, pl.ds(NS * RPT, N_NODES - NS * RPT)])


def _edge_aggregate(h_all, st16, edge_index, mx16):
    mesh = plsc.VectorSubcoreMesh(core_axis_name="c", subcore_axis_name="s")
    cp = pltpu.CompilerParams()
    fields = pltpu.CompilerParams.__dataclass_fields__
    if "needs_layout_passes" in fields:
        cp = dataclasses.replace(cp, needs_layout_passes=False)
    if "use_tc_tiling_on_sc" in fields:
        cp = dataclasses.replace(cp, use_tc_tiling_on_sc=False)
    run = pl.kernel(
        _edge_body,
        out_type=jax.ShapeDtypeStruct((NC, N_NODES, ROW), jnp.float32),
        mesh=mesh,
        compiler_params=cp,
        scratch_types=[
            pltpu.VMEM((2, CHUNK), jnp.int32),
            pltpu.VMEM((2, CHUNK), jnp.int32),
            pltpu.VMEM((CHUNK, IN_F), jnp.float32),
            pltpu.VMEM((CHUNK, IN_F), jnp.float32),
            pltpu.VMEM((CHUNK, 16), jnp.float32),
            pltpu.VMEM((CHUNK, 16), jnp.float32),
            pltpu.VMEM((CHUNK, 16), jnp.float32),
            pltpu.VMEM((CHUNK, 16), jnp.float32),
            pltpu.VMEM((MSGA, ROW), jnp.float32),
            pltpu.VMEM((MSGB, ROW), jnp.float32),
            pltpu.VMEM((MSGA,), jnp.int32),
            pltpu.VMEM((MSGB,), jnp.int32),
            pltpu.VMEM((16,), jnp.float32),
            pltpu.VMEM_SHARED((N_NODES, ROW), jnp.float32),
            pltpu.SemaphoreType.DMA,
            pltpu.SemaphoreType.DMA,
            pltpu.SemaphoreType.DMA,
            pltpu.SemaphoreType.DMA,
            pltpu.SemaphoreType.DMA,
            pltpu.SemaphoreType.DMA,
        ],
    )
    return run(h_all, st16, edge_index, mx16)


def _fin_body(p_ref, o_ref):
    y = p_ref[0] + p_ref[1]
    cols = []
    for h in range(HEADS):
        d = y[:, IN_F + h:IN_F + h + 1] + 1e-16
        cols.append(y[:, h * 32:(h + 1) * 32] / d)
    r = jnp.concatenate(cols, axis=1)
    o_ref[...] = jnp.where(r > 0.0, r, jnp.exp(r) - 1.0)


def _finalize(partials):
    grid = N_NODES // BN
    return pl.pallas_call(
        _fin_body,
        grid=(grid,),
        in_specs=[
            pl.BlockSpec((2, BN, ROW), lambda i: (0, i, 0)),
        ],
        out_specs=pl.BlockSpec((BN, IN_F), lambda i: (i, 0)),
        out_shape=jax.ShapeDtypeStruct((N_NODES, IN_F), jnp.float32),
    )(partials)


def kernel(x, edge_index, W, a):
    wc = W.transpose(1, 0, 2).reshape(IN_F, HEADS * OUT_F)
    amat = jnp.zeros((IN_F, 2 * HEADS), jnp.float32)
    for h in range(HEADS):
        amat = amat.at[h * OUT_F:(h + 1) * OUT_F, h].set(a[h, :OUT_F])
        amat = amat.at[h * OUT_F:(h + 1) * OUT_F, HEADS + h].set(a[h, OUT_F:])

    h_all, st16, mx = _project(x, wc, amat)
    mx16 = jnp.concatenate([mx[:, 0], jnp.zeros((8,), jnp.float32)])

    partials = _edge_aggregate(h_all, st16, edge_index, mx16)
    return _finalize(partials)
